# baseline (speedup 1.0000x reference)
import functools

import jax
import jax.numpy as jnp
from jax import lax
from jax.experimental import pallas as pl
from jax.experimental.pallas import tpu as pltpu
from jax.experimental.pallas import tpu_sc as plsc

_C = 96
_NB = 256
_NL = 16
_NW = 32
_CH = 4608
_CPW = _C // _NW


def _tc_moments(a3d, RB):
    _, H, W = a3d.shape
    nsteps = H // RB

    def body(x_ref, mom_ref, bc_ref, mm_ref):
        i = pl.program_id(0)
        x = x_ref[...]
        ps = jnp.sum(jnp.sum(x, axis=2), axis=1, keepdims=True)
        ps2 = jnp.sum(jnp.sum(x * x, axis=2), axis=1, keepdims=True)
        pmn = jnp.min(jnp.min(x, axis=2), axis=1, keepdims=True)
        pmx = jnp.max(jnp.max(x, axis=2), axis=1, keepdims=True)

        @pl.when(i == 0)
        def _():
            mom_ref[...] = jnp.concatenate((ps, ps2), axis=1)
            mm_ref[...] = jnp.concatenate((pmn, pmx), axis=1)

        @pl.when(i > 0)
        def _():
            mom_ref[...] = mom_ref[...] + jnp.concatenate((ps, ps2), axis=1)
            mm = mm_ref[...]
            mm_ref[...] = jnp.concatenate((jnp.minimum(mm[:, 0:1], pmn),
                                           jnp.maximum(mm[:, 1:2], pmx)), axis=1)

        @pl.when(i == nsteps - 1)
        def _():
            mm = mm_ref[...]
            dmn = mm[:, 0:1]
            scalec = jnp.float32(_NB) / jnp.maximum(mm[:, 1:2] - dmn,
                                                    jnp.float32(1e-30))
            bc_ref[...] = jnp.concatenate(
                (jnp.broadcast_to(dmn, (_C, _NL)),
                 jnp.broadcast_to(scalec, (_C, _NL))), axis=1)

    return pl.pallas_call(
        body,
        grid=(nsteps,),
        in_specs=[pl.BlockSpec((_C, RB, W), lambda i: (0, i, 0))],
        out_specs=[pl.BlockSpec((_C, 2), lambda i: (0, 0)),
                   pl.BlockSpec((_C, 2 * _NL), lambda i: (0, 0))],
        out_shape=[jax.ShapeDtypeStruct((_C, 2), jnp.float32),
                   jax.ShapeDtypeStruct((_C, 2 * _NL), jnp.float32)],
        scratch_shapes=[pltpu.VMEM((_C, 2), jnp.float32)],
    )(a3d)


def _sc_histograms(a0, a1, a2, bc0, bc1, bc2, chan_base):
    mesh = plsc.VectorSubcoreMesh(core_axis_name="c", subcore_axis_name="s",
                                  num_cores=1)
    half = _C // 2
    out_type = []
    for _ in range(3):
        out_type += [
            jax.ShapeDtypeStruct((half, _NB), jnp.float32),
            jax.ShapeDtypeStruct((half, _NB), jnp.float32),
        ]

    @functools.partial(
        pl.kernel,
        mesh=mesh,
        out_type=out_type,
        compiler_params=pltpu.CompilerParams(needs_layout_passes=False),
        scratch_types=[
            pltpu.VMEM((16, 384), jnp.float32),
            pltpu.VMEM((16, 384), jnp.float32),
            pltpu.VMEM((32, 192), jnp.float32),
            pltpu.VMEM((32, 192), jnp.float32),
            pltpu.VMEM((48, 96), jnp.float32),
            pltpu.VMEM((48, 96), jnp.float32),
            pltpu.VMEM((_NB * _NL,), jnp.float32),
            pltpu.VMEM((_NB * _NL,), jnp.float32),
            pltpu.VMEM((_NB * _NL,), jnp.float32),
            pltpu.VMEM((_NB * _NL,), jnp.float32),
            pltpu.VMEM((_NB * _NL,), jnp.float32),
            pltpu.VMEM((_NB * _NL,), jnp.float32),
            pltpu.VMEM((_NB * _NL,), jnp.float32),
            pltpu.VMEM((_NB * _NL,), jnp.float32),
            pltpu.VMEM((_NB,), jnp.float32),
            pltpu.VMEM((_NB,), jnp.float32),
            pltpu.VMEM((2 * _NL,), jnp.float32),
            pltpu.SemaphoreType.DMA,
            pltpu.SemaphoreType.DMA,
        ],
    )
    def k(a0h, a1h, a2h, bc0h, bc1h, bc2h,
          c0h, s0h, c1h, s1h, c2h, s2h,
          b0a, b0b, b1a, b1b, b2a, b2b,
          cv0, cv1, cv2, cv3, sv0, sv1, sv2, sv3,
          mc_v, ms_v, bc_v, sem_a, sem_b):
        cnt_vs = (cv0, cv1, cv2, cv3)
        sum_vs = (sv0, sv1, sv2, sv3)
        wid = lax.axis_index("s")
        lanes = lax.broadcasted_iota(jnp.int32, (_NL,), 0)
        laneoff = lanes * _NB
        ones = jnp.ones((_NL,), jnp.float32)
        zeros = jnp.zeros((_NL,), jnp.float32)

        def do_level(ah, bch, ch_out, sh_out, H, W, R, bufs):
            nchunks = H // R
            vpr = W // _NL

            def per_channel(t, _):
                ch = wid + 16 * t
                gch = chan_base + ch

                pltpu.sync_copy(bch.at[gch], bc_v)
                dmn = bc_v[pl.ds(0, _NL)]
                scale = bc_v[pl.ds(_NL, _NL)]

                sems = (sem_a, sem_b)
                pltpu.async_copy(ah.at[gch, pl.ds(0, R), :], bufs[0], sem_a)
                pltpu.async_copy(ah.at[gch, pl.ds(R, R), :], bufs[1], sem_b)

                nbm1 = jnp.full((_NL,), jnp.float32(_NB - 1))

                def scatter_buf(buf):
                    @plsc.parallel_loop(0, R, step=1, unroll=4)
                    def _p2_row(r):
                        for u in range(vpr):
                            x = buf[r, pl.ds(u * _NL, _NL)]
                            t_ = jnp.minimum((x - dmn) * scale, nbm1)
                            addr = laneoff + t_.astype(jnp.int32)
                            plsc.addupdate_scatter(cnt_vs[u % 4], [addr], ones)
                            plsc.addupdate_scatter(sum_vs[u % 4], [addr], x)

                def p2_pair(i2, _c):
                    for b in range(2):
                        i = i2 * 2 + b
                        pltpu.make_async_copy(
                            ah.at[gch, pl.ds(i * R, R), :], bufs[b],
                            sems[b]).wait()
                        scatter_buf(bufs[b])

                        @pl.when(i + 2 < nchunks)
                        def _refill():
                            pltpu.async_copy(
                                ah.at[gch, pl.ds((i + 2) * R, R), :],
                                bufs[b], sems[b])
                    return 0

                lax.fori_loop(0, nchunks // 2, p2_pair, 0)

                @plsc.parallel_loop(0, _NB, step=_NL)
                def _merge_body(gi):
                    off = pl.multiple_of(gi, _NL)
                    acc_c = zeros
                    acc_s = zeros
                    for r in range(4):
                        for l in range(_NL):
                            acc_c = acc_c + cnt_vs[r][pl.ds(l * _NB + off, _NL)]
                            cnt_vs[r][pl.ds(l * _NB + off, _NL)] = zeros
                            acc_s = acc_s + sum_vs[r][pl.ds(l * _NB + off, _NL)]
                            sum_vs[r][pl.ds(l * _NB + off, _NL)] = zeros
                    mc_v[pl.ds(off, _NL)] = acc_c
                    ms_v[pl.ds(off, _NL)] = acc_s
                pltpu.sync_copy(mc_v, ch_out.at[ch])
                pltpu.sync_copy(ms_v, sh_out.at[ch])
                return 0

            lax.fori_loop(0, _CPW, per_channel, 0)

        @plsc.parallel_loop(0, _NB * _NL, step=4 * _NL)
        def _zero_body(j):
            off = pl.multiple_of(j, _NL)
            for u in range(4):
                for r in range(4):
                    cnt_vs[r][pl.ds(off + u * _NL, _NL)] = zeros
                    sum_vs[r][pl.ds(off + u * _NL, _NL)] = zeros

        do_level(a0h, bc0h, c0h, s0h, 384, 384, 16, (b0a, b0b))
        do_level(a1h, bc1h, c1h, s1h, 192, 192, 32, (b1a, b1b))
        do_level(a2h, bc2h, c2h, s2h, 96, 96, 48, (b2a, b2b))

    return k(a0, a1, a2, bc0, bc1, bc2)


def _tc_finalize_body(Ns, G, nsteps,
                      h0, ht0, c0, s0, m0, n0, x0,
                      h1, ht1, c1, s1, m1, n1, x1,
                      h2, ht2, c2, s2, m2, n2, x2,
                      wref, out, acc):
    step = pl.program_id(0)
    f32 = jnp.float32

    bi2 = lax.broadcasted_iota(jnp.int32, (256, 256), 0)
    bj2 = lax.broadcasted_iota(jnp.int32, (256, 256), 1)
    lt256 = jnp.where(bj2 <= bi2, f32(1), f32(0))
    ut256 = jnp.where(bi2 <= bj2, f32(1), f32(0))
    qi = lax.broadcasted_iota(jnp.int32, (_NB, _NB), 0)
    qj = lax.broadcasted_iota(jnp.int32, (_NB, _NB), 1)
    ut512 = jnp.where(qi <= qj, f32(1), f32(0))

    brow = lax.broadcasted_iota(jnp.int32, (1, 256), 1)
    bmask_row = jnp.where(brow < 255, f32(1), f32(0))
    wts_row = 2.0 * brow.astype(f32) + 1.0
    bcol = lax.broadcasted_iota(jnp.int32, (256, 1), 0)
    bmask_col_b = bcol < 255
    ones_row = jnp.ones((1, 256), f32)
    hi = lax.Precision.HIGHEST

    @pl.when(step == 0)
    def _init():
        acc[0] = f32(0)
        acc[1] = f32(0)
        acc[2] = f32(0)

    parts = []
    for (h, ht, cr, sr, mr, pmnr, pmxr, N) in (
            (h0, ht0, c0, s0, m0, n0, x0, Ns[0]),
            (h1, ht1, c1, s1, m1, n1, x1, Ns[1]),
            (h2, ht2, c2, s2, m2, n2, x2, Ns[2])):
        Nf = f32(N)
        hist_blk = h[...]
        cdf_r = jax.lax.dot_general(hist_blk, ut256, (((1,), (0,)), ((), ())),
                                    precision=hi)
        cdf_r = cdf_r / cdf_r[:, 255:256]
        histT_blk = ht[0]
        cdf_c = jax.lax.dot_general(lt256, histT_blk, (((1,), (0,)), ((), ())),
                                    precision=hi)
        cdf_c = cdf_c / cdf_c[255:256, :]

        def rank_thresholds(cdf):
            k0 = jnp.clip((cdf * Nf).astype(jnp.int32), 0, N).astype(f32)

            def g(i):
                ok = ((i / Nf) <= cdf) & (i <= Nf)
                return jnp.where((i <= 0.0) | ok, f32(1), f32(0))

            k = (k0 - 2.0) + g(k0 - 1.0) + g(k0) + g(k0 + 1.0) + g(k0 + 2.0)
            return jnp.clip(k, 0.0, Nf)

        k_rows = rank_thresholds(cdf_r)
        k_cols = rank_thresholds(cdf_c)

        counts_blk = cr[...]
        sums_blk = sr[...]
        cc_rows = jax.lax.dot_general(counts_blk, ut512, (((1,), (0,)), ((), ())),
                                      precision=hi)
        cs_rows = jax.lax.dot_general(sums_blk, ut512, (((1,), (0,)), ((), ())),
                                      precision=hi)

        part = f32(0)
        for gch in range(G):
            k_row = k_rows[gch:gch + 1, :]
            k_col = k_cols[:, gch:gch + 1]
            cnt_row = counts_blk[gch:gch + 1, :]
            sum_row = sums_blk[gch:gch + 1, :]
            cc = cc_rows[gch:gch + 1, :]
            cs = cs_rows[gch:gch + 1, :]
            ccp = cc - cnt_row
            csp = cs - sum_row
            rmean = sum_row / jnp.maximum(cnt_row, f32(1))

            cond = (ccp < k_col) & (cc >= k_col) & bmask_col_b
            at = jnp.where(cond, f32(1), f32(0))
            lhs = jnp.concatenate((ones_row, k_row), axis=0)
            res = jax.lax.dot_general(lhs, at, (((1,), (0,)), ((), ())),
                                      precision=hi)
            rowsum_at = res[0:1, :]
            atk = res[1:2, :]
            ssum = jnp.sum(csp * rowsum_at + rmean * atk
                           - ccp * rmean * rowsum_at)
            ksum = jnp.sum(k_row * bmask_row)
            kw = jnp.sum(k_row * wts_row * bmask_row)

            a_ = mr[gch, 0]
            b_ = mr[gch, 1]
            pmn = pmnr[gch, 0]
            pmx = pmxr[gch, 0]
            s_ = pmx - pmn
            cross_sum = 255.0 * a_ - ssum
            e1 = s_ * cross_sum / 255.0 + pmn * a_
            sumv = (255.0 * Nf - ksum) / 255.0
            sumv2 = (Nf * (255.0 * 255.0) - kw) / (255.0 * 255.0)
            e2 = s_ * s_ * sumv2 + 2.0 * s_ * pmn * sumv + Nf * pmn * pmn
            part = part + (b_ - 2.0 * e1 + e2)
        parts.append(part)

    acc[0] = acc[0] + parts[0]
    acc[1] = acc[1] + parts[1]
    acc[2] = acc[2] + parts[2]

    @pl.when(step == nsteps - 1)
    def _fin():
        w0 = wref[0, 0]
        w1 = wref[0, 1]
        w2 = wref[0, 2]
        val = (w0 * acc[0] / f32(_C * Ns[0])
               + w1 * acc[1] / f32(_C * Ns[1])
               + w2 * acc[2] / f32(_C * Ns[2]))
        out[...] = jnp.broadcast_to(val, (1, 1))


def _tc_finalize(Ns, hists, histTs, cnts, sums, moms, pmns, pmxs, weights):
    G = 8
    nsteps = _C // G

    def lvl_specs():
        return [
            pl.BlockSpec((G, 256), lambda s: (s, 0)),
            pl.BlockSpec((1, 256, G), lambda s: (s, 0, 0)),
            pl.BlockSpec((G, _NB), lambda s: (s, 0)),
            pl.BlockSpec((G, _NB), lambda s: (s, 0)),
            pl.BlockSpec((G, 2), lambda s: (s, 0)),
            pl.BlockSpec((G, 1), lambda s: (s, 0)),
            pl.BlockSpec((G, 1), lambda s: (s, 0)),
        ]

    in_specs = lvl_specs() + lvl_specs() + lvl_specs() + [
        pl.BlockSpec((1, 3), lambda s: (0, 0)),
    ]
    args = []
    for i in range(3):
        args += [hists[i], histTs[i], cnts[i], sums[i], moms[i],
                 pmns[i], pmxs[i]]
    args.append(weights)

    return pl.pallas_call(
        functools.partial(_tc_finalize_body, Ns, G, nsteps),
        grid=(nsteps,),
        in_specs=in_specs,
        out_specs=pl.BlockSpec((1, 1), lambda s: (0, 0)),
        out_shape=jax.ShapeDtypeStruct((1, 1), jnp.float32),
        scratch_shapes=[pltpu.SMEM((3,), jnp.float32)],
    )(*args)


def kernel(act0, act1, act2, hist0, hist1, hist2, min0, min1, min2,
           max0, max1, max2, mip_weights, bins):
    del bins
    Ns = (act0.shape[2] * act0.shape[3],
          act1.shape[2] * act1.shape[3],
          act2.shape[2] * act2.shape[3])

    a0 = act0.reshape(act0.shape[1:])
    a1 = act1.reshape(act1.shape[1:])
    a2 = act2.reshape(act2.shape[1:])

    moms = []
    bcs = []
    for a3d, RB in ((a0, 48), (a1, 96), (a2, 96)):
        mom, bc = _tc_moments(a3d, RB)
        moms.append(mom)
        bcs.append(bc)

    ha = _sc_histograms(a0, a1, a2, bcs[0], bcs[1], bcs[2], 0)
    hb = _sc_histograms(a0, a1, a2, bcs[0], bcs[1], bcs[2], _C // 2)
    (c0, s0, c1, s1, c2, s2) = tuple(
        jnp.concatenate((xa, xb), axis=0) for xa, xb in zip(ha, hb))

    hists = (hist0, hist1, hist2)
    histTs = tuple(h.reshape(12, 8, 256).transpose(0, 2, 1)
                   for h in (hist0, hist1, hist2))
    pmns = (min0.reshape(_C, 1), min1.reshape(_C, 1), min2.reshape(_C, 1))
    pmxs = (max0.reshape(_C, 1), max1.reshape(_C, 1), max2.reshape(_C, 1))
    out = _tc_finalize(Ns, hists, histTs, (c0, c1, c2), (s0, s1, s2),
                       moms, pmns, pmxs, mip_weights.reshape(1, 3))
    return out[0, 0]


# speedup vs baseline: 1.6265x; 1.6265x over previous
import functools

import jax
import jax.numpy as jnp
from jax import lax
from jax.experimental import pallas as pl
from jax.experimental.pallas import tpu as pltpu
from jax.experimental.pallas import tpu_sc as plsc

_C = 96
_NB = 256
_NL = 16
_NW = 32
_CH = 4608
_CPW = _C // _NW


def _tc_moments(a3d, RB):
    _, H, W = a3d.shape
    nsteps = H // RB

    def body(x_ref, mom_ref, bc_ref, mm_ref):
        i = pl.program_id(0)
        x = x_ref[...]
        ps = jnp.sum(jnp.sum(x, axis=2), axis=1, keepdims=True)
        ps2 = jnp.sum(jnp.sum(x * x, axis=2), axis=1, keepdims=True)
        pmn = jnp.min(jnp.min(x, axis=2), axis=1, keepdims=True)
        pmx = jnp.max(jnp.max(x, axis=2), axis=1, keepdims=True)

        @pl.when(i == 0)
        def _():
            mom_ref[...] = jnp.concatenate((ps, ps2), axis=1)
            mm_ref[...] = jnp.concatenate((pmn, pmx), axis=1)

        @pl.when(i > 0)
        def _():
            mom_ref[...] = mom_ref[...] + jnp.concatenate((ps, ps2), axis=1)
            mm = mm_ref[...]
            mm_ref[...] = jnp.concatenate((jnp.minimum(mm[:, 0:1], pmn),
                                           jnp.maximum(mm[:, 1:2], pmx)), axis=1)

        @pl.when(i == nsteps - 1)
        def _():
            mm = mm_ref[...]
            dmn = mm[:, 0:1]
            scalec = jnp.float32(_NB) / jnp.maximum(mm[:, 1:2] - dmn,
                                                    jnp.float32(1e-30))
            bc_ref[...] = jnp.concatenate(
                (jnp.broadcast_to(dmn, (_C, _NL)),
                 jnp.broadcast_to(scalec, (_C, _NL))), axis=1)

    return pl.pallas_call(
        body,
        grid=(nsteps,),
        in_specs=[pl.BlockSpec((_C, RB, W), lambda i: (0, i, 0))],
        out_specs=[pl.BlockSpec((_C, 2), lambda i: (0, 0)),
                   pl.BlockSpec((_C, 2 * _NL), lambda i: (0, 0))],
        out_shape=[jax.ShapeDtypeStruct((_C, 2), jnp.float32),
                   jax.ShapeDtypeStruct((_C, 2 * _NL), jnp.float32)],
        scratch_shapes=[pltpu.VMEM((_C, 2), jnp.float32)],
    )(a3d)


def _sc_histograms(a0, a1, a2, bc0, bc1, bc2, chan_base):
    mesh = plsc.VectorSubcoreMesh(core_axis_name="c", subcore_axis_name="s")
    out_type = []
    for _ in range(3):
        out_type += [
            jax.ShapeDtypeStruct((_C, _NB), jnp.float32),
            jax.ShapeDtypeStruct((_C, _NB), jnp.float32),
        ]

    @functools.partial(
        pl.kernel,
        mesh=mesh,
        out_type=out_type,
        compiler_params=pltpu.CompilerParams(needs_layout_passes=False),
        scratch_types=[
            pltpu.VMEM((16, 384), jnp.float32),
            pltpu.VMEM((16, 384), jnp.float32),
            pltpu.VMEM((32, 192), jnp.float32),
            pltpu.VMEM((32, 192), jnp.float32),
            pltpu.VMEM((48, 96), jnp.float32),
            pltpu.VMEM((48, 96), jnp.float32),
            pltpu.VMEM((_NB * _NL,), jnp.float32),
            pltpu.VMEM((_NB * _NL,), jnp.float32),
            pltpu.VMEM((_NB * _NL,), jnp.float32),
            pltpu.VMEM((_NB * _NL,), jnp.float32),
            pltpu.VMEM((_NB * _NL,), jnp.float32),
            pltpu.VMEM((_NB * _NL,), jnp.float32),
            pltpu.VMEM((_NB * _NL,), jnp.float32),
            pltpu.VMEM((_NB * _NL,), jnp.float32),
            pltpu.VMEM((_NB,), jnp.float32),
            pltpu.VMEM((_NB,), jnp.float32),
            pltpu.VMEM((2 * _NL,), jnp.float32),
            pltpu.SemaphoreType.DMA,
            pltpu.SemaphoreType.DMA,
        ],
    )
    def k(a0h, a1h, a2h, bc0h, bc1h, bc2h,
          c0h, s0h, c1h, s1h, c2h, s2h,
          b0a, b0b, b1a, b1b, b2a, b2b,
          cv0, cv1, cv2, cv3, sv0, sv1, sv2, sv3,
          mc_v, ms_v, bc_v, sem_a, sem_b):
        cnt_vs = (cv0, cv1, cv2, cv3)
        sum_vs = (sv0, sv1, sv2, sv3)
        wid = lax.axis_index("s") * 2 + lax.axis_index("c")
        lanes = lax.broadcasted_iota(jnp.int32, (_NL,), 0)
        laneoff = lanes * _NB
        ones = jnp.ones((_NL,), jnp.float32)
        zeros = jnp.zeros((_NL,), jnp.float32)

        def do_level(ah, bch, ch_out, sh_out, H, W, R, bufs):
            nchunks = H // R
            vpr = W // _NL

            def per_channel(t, _):
                ch = wid + _NW * t
                gch = ch

                pltpu.sync_copy(bch.at[gch], bc_v)
                dmn = bc_v[pl.ds(0, _NL)]
                scale = bc_v[pl.ds(_NL, _NL)]

                sems = (sem_a, sem_b)
                pltpu.async_copy(ah.at[gch, pl.ds(0, R), :], bufs[0], sem_a)
                pltpu.async_copy(ah.at[gch, pl.ds(R, R), :], bufs[1], sem_b)

                nbm1 = jnp.full((_NL,), jnp.float32(_NB - 1))

                def scatter_buf(buf):
                    @plsc.parallel_loop(0, R, step=1, unroll=4)
                    def _p2_row(r):
                        for u in range(vpr):
                            x = buf[r, pl.ds(u * _NL, _NL)]
                            t_ = jnp.minimum((x - dmn) * scale, nbm1)
                            addr = laneoff + t_.astype(jnp.int32)
                            plsc.addupdate_scatter(cnt_vs[u % 4], [addr], ones)
                            plsc.addupdate_scatter(sum_vs[u % 4], [addr], x)

                def p2_pair(i2, _c):
                    for b in range(2):
                        i = i2 * 2 + b
                        pltpu.make_async_copy(
                            ah.at[gch, pl.ds(i * R, R), :], bufs[b],
                            sems[b]).wait()
                        scatter_buf(bufs[b])

                        @pl.when(i + 2 < nchunks)
                        def _refill():
                            pltpu.async_copy(
                                ah.at[gch, pl.ds((i + 2) * R, R), :],
                                bufs[b], sems[b])
                    return 0

                lax.fori_loop(0, nchunks // 2, p2_pair, 0)

                @plsc.parallel_loop(0, _NB, step=_NL)
                def _merge_body(gi):
                    off = pl.multiple_of(gi, _NL)
                    acc_c = zeros
                    acc_s = zeros
                    for r in range(4):
                        for l in range(_NL):
                            acc_c = acc_c + cnt_vs[r][pl.ds(l * _NB + off, _NL)]
                            cnt_vs[r][pl.ds(l * _NB + off, _NL)] = zeros
                            acc_s = acc_s + sum_vs[r][pl.ds(l * _NB + off, _NL)]
                            sum_vs[r][pl.ds(l * _NB + off, _NL)] = zeros
                    mc_v[pl.ds(off, _NL)] = acc_c
                    ms_v[pl.ds(off, _NL)] = acc_s
                pltpu.sync_copy(mc_v, ch_out.at[ch])
                pltpu.sync_copy(ms_v, sh_out.at[ch])
                return 0

            lax.fori_loop(0, _CPW, per_channel, 0)

        @plsc.parallel_loop(0, _NB * _NL, step=4 * _NL)
        def _zero_body(j):
            off = pl.multiple_of(j, _NL)
            for u in range(4):
                for r in range(4):
                    cnt_vs[r][pl.ds(off + u * _NL, _NL)] = zeros
                    sum_vs[r][pl.ds(off + u * _NL, _NL)] = zeros

        do_level(a0h, bc0h, c0h, s0h, 384, 384, 16, (b0a, b0b))
        do_level(a1h, bc1h, c1h, s1h, 192, 192, 32, (b1a, b1b))
        do_level(a2h, bc2h, c2h, s2h, 96, 96, 48, (b2a, b2b))

    return k(a0, a1, a2, bc0, bc1, bc2)


def _tc_finalize_body(Ns, G, nsteps,
                      h0, ht0, c0, s0, m0, n0, x0,
                      h1, ht1, c1, s1, m1, n1, x1,
                      h2, ht2, c2, s2, m2, n2, x2,
                      wref, out, acc):
    step = pl.program_id(0)
    f32 = jnp.float32

    bi2 = lax.broadcasted_iota(jnp.int32, (256, 256), 0)
    bj2 = lax.broadcasted_iota(jnp.int32, (256, 256), 1)
    lt256 = jnp.where(bj2 <= bi2, f32(1), f32(0))
    ut256 = jnp.where(bi2 <= bj2, f32(1), f32(0))
    qi = lax.broadcasted_iota(jnp.int32, (_NB, _NB), 0)
    qj = lax.broadcasted_iota(jnp.int32, (_NB, _NB), 1)
    ut512 = jnp.where(qi <= qj, f32(1), f32(0))

    brow = lax.broadcasted_iota(jnp.int32, (1, 256), 1)
    bmask_row = jnp.where(brow < 255, f32(1), f32(0))
    wts_row = 2.0 * brow.astype(f32) + 1.0
    bcol = lax.broadcasted_iota(jnp.int32, (256, 1), 0)
    bmask_col_b = bcol < 255
    ones_row = jnp.ones((1, 256), f32)
    hi = lax.Precision.HIGHEST

    @pl.when(step == 0)
    def _init():
        acc[0] = f32(0)
        acc[1] = f32(0)
        acc[2] = f32(0)

    parts = []
    for (h, ht, cr, sr, mr, pmnr, pmxr, N) in (
            (h0, ht0, c0, s0, m0, n0, x0, Ns[0]),
            (h1, ht1, c1, s1, m1, n1, x1, Ns[1]),
            (h2, ht2, c2, s2, m2, n2, x2, Ns[2])):
        Nf = f32(N)
        hist_blk = h[...]
        cdf_r = jax.lax.dot_general(hist_blk, ut256, (((1,), (0,)), ((), ())),
                                    precision=hi)
        cdf_r = cdf_r / cdf_r[:, 255:256]
        histT_blk = ht[0]
        cdf_c = jax.lax.dot_general(lt256, histT_blk, (((1,), (0,)), ((), ())),
                                    precision=hi)
        cdf_c = cdf_c / cdf_c[255:256, :]

        def rank_thresholds(cdf):
            k0 = jnp.clip((cdf * Nf).astype(jnp.int32), 0, N).astype(f32)

            def g(i):
                ok = ((i / Nf) <= cdf) & (i <= Nf)
                return jnp.where((i <= 0.0) | ok, f32(1), f32(0))

            k = (k0 - 2.0) + g(k0 - 1.0) + g(k0) + g(k0 + 1.0) + g(k0 + 2.0)
            return jnp.clip(k, 0.0, Nf)

        k_rows = rank_thresholds(cdf_r)
        k_cols = rank_thresholds(cdf_c)

        counts_blk = cr[...]
        sums_blk = sr[...]
        cc_rows = jax.lax.dot_general(counts_blk, ut512, (((1,), (0,)), ((), ())),
                                      precision=hi)
        cs_rows = jax.lax.dot_general(sums_blk, ut512, (((1,), (0,)), ((), ())),
                                      precision=hi)

        part = f32(0)
        for gch in range(G):
            k_row = k_rows[gch:gch + 1, :]
            k_col = k_cols[:, gch:gch + 1]
            cnt_row = counts_blk[gch:gch + 1, :]
            sum_row = sums_blk[gch:gch + 1, :]
            cc = cc_rows[gch:gch + 1, :]
            cs = cs_rows[gch:gch + 1, :]
            ccp = cc - cnt_row
            csp = cs - sum_row
            rmean = sum_row / jnp.maximum(cnt_row, f32(1))

            cond = (ccp < k_col) & (cc >= k_col) & bmask_col_b
            at = jnp.where(cond, f32(1), f32(0))
            lhs = jnp.concatenate((ones_row, k_row), axis=0)
            res = jax.lax.dot_general(lhs, at, (((1,), (0,)), ((), ())),
                                      precision=hi)
            rowsum_at = res[0:1, :]
            atk = res[1:2, :]
            ssum = jnp.sum(csp * rowsum_at + rmean * atk
                           - ccp * rmean * rowsum_at)
            ksum = jnp.sum(k_row * bmask_row)
            kw = jnp.sum(k_row * wts_row * bmask_row)

            a_ = mr[gch, 0]
            b_ = mr[gch, 1]
            pmn = pmnr[gch, 0]
            pmx = pmxr[gch, 0]
            s_ = pmx - pmn
            cross_sum = 255.0 * a_ - ssum
            e1 = s_ * cross_sum / 255.0 + pmn * a_
            sumv = (255.0 * Nf - ksum) / 255.0
            sumv2 = (Nf * (255.0 * 255.0) - kw) / (255.0 * 255.0)
            e2 = s_ * s_ * sumv2 + 2.0 * s_ * pmn * sumv + Nf * pmn * pmn
            part = part + (b_ - 2.0 * e1 + e2)
        parts.append(part)

    acc[0] = acc[0] + parts[0]
    acc[1] = acc[1] + parts[1]
    acc[2] = acc[2] + parts[2]

    @pl.when(step == nsteps - 1)
    def _fin():
        w0 = wref[0, 0]
        w1 = wref[0, 1]
        w2 = wref[0, 2]
        val = (w0 * acc[0] / f32(_C * Ns[0])
               + w1 * acc[1] / f32(_C * Ns[1])
               + w2 * acc[2] / f32(_C * Ns[2]))
        out[...] = jnp.broadcast_to(val, (1, 1))


def _tc_finalize(Ns, hists, histTs, cnts, sums, moms, pmns, pmxs, weights):
    G = 8
    nsteps = _C // G

    def lvl_specs():
        return [
            pl.BlockSpec((G, 256), lambda s: (s, 0)),
            pl.BlockSpec((1, 256, G), lambda s: (s, 0, 0)),
            pl.BlockSpec((G, _NB), lambda s: (s, 0)),
            pl.BlockSpec((G, _NB), lambda s: (s, 0)),
            pl.BlockSpec((G, 2), lambda s: (s, 0)),
            pl.BlockSpec((G, 1), lambda s: (s, 0)),
            pl.BlockSpec((G, 1), lambda s: (s, 0)),
        ]

    in_specs = lvl_specs() + lvl_specs() + lvl_specs() + [
        pl.BlockSpec((1, 3), lambda s: (0, 0)),
    ]
    args = []
    for i in range(3):
        args += [hists[i], histTs[i], cnts[i], sums[i], moms[i],
                 pmns[i], pmxs[i]]
    args.append(weights)

    return pl.pallas_call(
        functools.partial(_tc_finalize_body, Ns, G, nsteps),
        grid=(nsteps,),
        in_specs=in_specs,
        out_specs=pl.BlockSpec((1, 1), lambda s: (0, 0)),
        out_shape=jax.ShapeDtypeStruct((1, 1), jnp.float32),
        scratch_shapes=[pltpu.SMEM((3,), jnp.float32)],
    )(*args)


def kernel(act0, act1, act2, hist0, hist1, hist2, min0, min1, min2,
           max0, max1, max2, mip_weights, bins):
    del bins
    Ns = (act0.shape[2] * act0.shape[3],
          act1.shape[2] * act1.shape[3],
          act2.shape[2] * act2.shape[3])

    a0 = act0.reshape(act0.shape[1:])
    a1 = act1.reshape(act1.shape[1:])
    a2 = act2.reshape(act2.shape[1:])

    moms = []
    bcs = []
    for a3d, RB in ((a0, 48), (a1, 96), (a2, 96)):
        mom, bc = _tc_moments(a3d, RB)
        moms.append(mom)
        bcs.append(bc)

    (c0, s0, c1, s1, c2, s2) = _sc_histograms(
        a0, a1, a2, bcs[0], bcs[1], bcs[2], 0)

    hists = (hist0, hist1, hist2)
    histTs = tuple(h.reshape(12, 8, 256).transpose(0, 2, 1)
                   for h in (hist0, hist1, hist2))
    pmns = (min0.reshape(_C, 1), min1.reshape(_C, 1), min2.reshape(_C, 1))
    pmxs = (max0.reshape(_C, 1), max1.reshape(_C, 1), max2.reshape(_C, 1))
    out = _tc_finalize(Ns, hists, histTs, (c0, c1, c2), (s0, s1, s2),
                       moms, pmns, pmxs, mip_weights.reshape(1, 3))
    return out[0, 0]
